# Initial kernel scaffold; baseline (speedup 1.0000x reference)
import jax
import jax.numpy as jnp
from jax.experimental import pallas as pl


def kernel(pred_logits, pred_boxes, orig_target_sizes):
    raise NotImplementedError("write your pallas kernel here")


# speedup vs baseline: 6.6871x; 6.6871x over previous
import functools

import jax
import jax.numpy as jnp
from jax.experimental import pallas as pl
from jax.experimental.pallas import tpu as pltpu

B = 16
NQ = 20000
NCLS = 80
FLAT = NQ * NCLS
NROW = 12500
NLANE = 128
NGRP = FLAT // 8
KGRP = 512
K1 = 320
NCAND = KGRP * 8
NSLOT = 512
KOUT = 300
BOXROW = NQ * 4 // NLANE


def _f32(x):
    return x.astype(jnp.float32)


def _amax_body(x_ref, o_ref):
    x = x_ref[0]
    w = jnp.maximum(x, jnp.concatenate([x[:, 4:], x[:, :4]], axis=1))
    w = jnp.maximum(w, jnp.concatenate([w[:, 2:], w[:, :2]], axis=1))
    w = jnp.maximum(w, jnp.concatenate([w[:, 1:], w[:, :1]], axis=1))
    li = jax.lax.broadcasted_iota(jnp.int32, (NLANE, 16), 0)
    ci = jax.lax.broadcasted_iota(jnp.int32, (NLANE, 16), 1)
    sel = _f32(li == 8 * ci)
    o_ref[0] = jax.lax.dot_general(
        w, sel, (((1,), (0,)), ((), ())), preferred_element_type=jnp.float32)


def _group_max(x_rows):
    return pl.pallas_call(
        _amax_body,
        grid=(B,),
        in_specs=[pl.BlockSpec((1, NROW, NLANE), lambda b: (b, 0, 0))],
        out_specs=pl.BlockSpec((1, NROW, 16), lambda b: (b, 0, 0)),
        out_shape=jax.ShapeDtypeStruct((B, NROW, 16), jnp.float32),
    )(x_rows)


def _gather_body(gsm, x_ref, gv_ref, bx_ref, oc_ref, ob_ref, rbuf, bbuf):
    b = pl.program_id(0)

    def body(i, _):
        g = gsm[b * KGRP + i]
        rbuf[pl.ds(i, 1), :] = x_ref[0, pl.ds(g // 16, 1), :]
        q = g // 10
        bbuf[pl.ds(i, 1), :] = bx_ref[0, pl.ds(q // 32, 1), :]
        return 0

    jax.lax.fori_loop(0, KGRP, body, 0)
    gcol = gv_ref[0]
    rows = rbuf[...]
    pcol = gcol % 16
    acc = rows[:, 0:8]
    for p in range(1, 16):
        acc = jnp.where(pcol == p, rows[:, 8 * p:8 * p + 8], acc)
    oc_ref[0] = acc
    brows = bbuf[...]
    qcol = (gcol // 10) % 32
    bacc = brows[:, 0:4]
    for p in range(1, 32):
        bacc = jnp.where(qcol == p, brows[:, 4 * p:4 * p + 4], bacc)
    ob_ref[0] = bacc


def _gather_candidates(x_rows, box_rows, gids):
    grid_spec = pltpu.PrefetchScalarGridSpec(
        num_scalar_prefetch=1,
        grid=(B,),
        in_specs=[
            pl.BlockSpec((1, NROW, NLANE), lambda b, g: (b, 0, 0)),
            pl.BlockSpec((1, KGRP, 1), lambda b, g: (b, 0, 0)),
            pl.BlockSpec((1, BOXROW, NLANE), lambda b, g: (b, 0, 0)),
        ],
        out_specs=[
            pl.BlockSpec((1, KGRP, 8), lambda b, g: (b, 0, 0)),
            pl.BlockSpec((1, KGRP, 4), lambda b, g: (b, 0, 0)),
        ],
        scratch_shapes=[
            pltpu.VMEM((KGRP, NLANE), jnp.float32),
            pltpu.VMEM((KGRP, NLANE), jnp.float32),
        ],
    )
    return pl.pallas_call(
        _gather_body,
        grid_spec=grid_spec,
        out_shape=[
            jax.ShapeDtypeStruct((B, KGRP, 8), jnp.float32),
            jax.ShapeDtypeStruct((B, KGRP, 4), jnp.float32),
        ],
    )(gids.reshape(-1), x_rows, gids.reshape(B, KGRP, 1), box_rows)


def _gather_rows_body(gsm, x_ref, o_ref):
    b = pl.program_id(0)

    def body(i, _):
        g = gsm[b * KGRP + i]
        o_ref[0, pl.ds(i, 1), :] = x_ref[0, pl.ds(g // 16, 1), :]
        return 0

    jax.lax.fori_loop(0, KGRP, body, 0)


def _debug_gather_rows(x_rows, gids):
    grid_spec = pltpu.PrefetchScalarGridSpec(
        num_scalar_prefetch=1,
        grid=(B,),
        in_specs=[pl.BlockSpec((1, NROW, NLANE), lambda b, g: (b, 0, 0))],
        out_specs=pl.BlockSpec((1, KGRP, NLANE), lambda b, g: (b, 0, 0)),
    )
    return pl.pallas_call(
        _gather_rows_body,
        grid_spec=grid_spec,
        out_shape=jax.ShapeDtypeStruct((B, KGRP, NLANE), jnp.float32),
    )(gids.reshape(-1), x_rows)


def _dot(a, b):
    return jax.lax.dot_general(
        a, b, (((1,), (0,)), ((), ())), preferred_element_type=jnp.float32)


def _dot_rt(a, b):
    return jax.lax.dot_general(
        a, b, (((1,), (1,)), ((), ())), preferred_element_type=jnp.float32)


def _bc(x, shape):
    return jnp.broadcast_to(x, shape)


def _select_body(s_ref, fi_ref, cb_ref, sc_ref, lab_ref, box_ref, sco_ref):
    s = s_ref[0]
    fi = fi_ref[0]
    w = jnp.maximum(s, jnp.concatenate([s[:, 4:], s[:, :4]], axis=1))
    w = jnp.maximum(w, jnp.concatenate([w[:, 2:], w[:, :2]], axis=1))
    w = jnp.maximum(w, jnp.concatenate([w[:, 1:], w[:, :1]], axis=1))
    li = jax.lax.broadcasted_iota(jnp.int32, (NCAND, KGRP), 0)
    ci = jax.lax.broadcasted_iota(jnp.int32, (NCAND, KGRP), 1)
    sel = _f32(li == 8 * ci)
    selt = _f32(
        jax.lax.broadcasted_iota(jnp.int32, (KGRP, NCAND), 1)
        == 8 * jax.lax.broadcasted_iota(jnp.int32, (KGRP, NCAND), 0))
    m_row = _dot(w, sel)
    m_col = _dot_rt(selt, w)
    ones_r512 = jnp.ones((1, KGRP), jnp.float32)
    ones_c512 = jnp.ones((KGRP, 1), jnp.float32)
    p1 = _f32(_dot(m_col, ones_r512) > _dot(ones_c512, m_row))
    cnt = jnp.sum(p1, axis=0, keepdims=True)
    t = jnp.min(jnp.where(cnt < K1, m_row, jnp.inf))
    keep = _f32(s >= t)
    pl.debug_print("DBG w_max_mil={}", (jnp.max(w) * 1000.0).astype(jnp.int32))
    pl.debug_print("DBG m_max_mil={}", (jnp.max(m_row) * 1000.0).astype(jnp.int32))
    pl.debug_print("DBG mcol_max_mil={}", (jnp.max(m_col) * 1000.0).astype(jnp.int32))
    pl.debug_print("DBG cnt_max={}", jnp.max(cnt).astype(jnp.int32))
    pl.debug_print("DBG t_mil={}", (t * 1000.0).astype(jnp.int32))
    pl.debug_print("DBG nkeep={}", jnp.sum(keep).astype(jnp.int32))
    cs = keep
    lane = jax.lax.broadcasted_iota(jnp.int32, (1, NCAND), 1)
    sh = 1
    while sh < NCAND:
        rolled = jnp.concatenate(
            [cs[:, NCAND - sh:], cs[:, :NCAND - sh]], axis=1)
        cs = cs + jnp.where(lane >= sh, rolled, 0.0)
        sh *= 2
    pos = cs - 1.0
    ones_cs = jnp.ones((NSLOT, 1), jnp.float32)
    pos2 = _dot(ones_cs, pos).astype(jnp.int32)
    keep2 = _dot(ones_cs, keep)
    slot2 = jax.lax.broadcasted_iota(jnp.int32, (NSLOT, NCAND), 0)
    oht = jnp.where(keep2 > 0.0, _f32(slot2 == pos2), 0.0)
    pl.debug_print("DBG cs_max={}", jnp.max(cs).astype(jnp.int32))
    pl.debug_print("DBG oht_sum={}", jnp.sum(oht).astype(jnp.int32))
    sc_row = _dot_rt(s, oht)
    sc_col = _dot_rt(oht, s)
    fi_row = _dot_rt(fi, oht)
    fi_col = _dot_rt(oht, fi)
    giota = _f32(jax.lax.broadcasted_iota(jnp.int32, (1, NCAND), 1) // 8)
    g_col = _dot_rt(oht, giota)
    gj = jax.lax.broadcasted_iota(jnp.int32, (NSLOT, KGRP), 1)
    oh2 = _f32(_dot(g_col, ones_r512).astype(jnp.int32) == gj)
    bslot = _dot(oh2, cb_ref[0])
    sc_c2 = _dot(sc_col, ones_r512)
    sc_r2 = _dot(ones_c512, sc_row)
    fi_c2 = _dot(fi_col, ones_r512)
    fi_r2 = _dot(ones_c512, fi_row)
    gt = _f32(sc_c2 > sc_r2)
    tie = jnp.where(sc_c2 == sc_r2, _f32(fi_c2 < fi_r2), 0.0)
    rank_row = jnp.sum(gt + tie, axis=0, keepdims=True)
    ones_ck = jnp.ones((KOUT, 1), jnp.float32)
    ro2 = jax.lax.broadcasted_iota(jnp.int32, (KOUT, NSLOT), 0)
    oh3t = _f32(_dot(ones_ck, rank_row).astype(jnp.int32) == ro2)
    pl.debug_print("DBG scrow_max_mil={}", (jnp.max(sc_row) * 1000.0).astype(jnp.int32))
    pl.debug_print("DBG rank_max={}", jnp.max(rank_row).astype(jnp.int32))
    pl.debug_print("DBG oh2_sum={}", jnp.sum(oh2).astype(jnp.int32))
    pl.debug_print("DBG oh3t_sum={}", jnp.sum(oh3t).astype(jnp.int32))
    sco_ref[0] = _dot_rt(sc_row, oh3t)
    lab_row = _f32(fi_row.astype(jnp.int32) % NCLS)
    lab_ref[0] = _dot_rt(lab_row, oh3t).astype(jnp.int32)
    raw = _dot(oh3t, bslot)
    cx, cy, ww, hh = raw[:, 0:1], raw[:, 1:2], raw[:, 2:3], raw[:, 3:4]
    xyxy = jnp.concatenate(
        [cx - 0.5 * ww, cy - 0.5 * hh, cx + 0.5 * ww, cy + 0.5 * hh], axis=1)
    box_ref[0] = xyxy * sc_ref[0]


def _select(s_t, fi_t, cand_boxes, scale):
    return pl.pallas_call(
        _select_body,
        grid=(B,),
        in_specs=[
            pl.BlockSpec((1, 1, NCAND), lambda b: (b, 0, 0)),
            pl.BlockSpec((1, 1, NCAND), lambda b: (b, 0, 0)),
            pl.BlockSpec((1, KGRP, 4), lambda b: (b, 0, 0)),
            pl.BlockSpec((1, 1, 4), lambda b: (b, 0, 0)),
        ],
        out_specs=[
            pl.BlockSpec((1, 1, KOUT), lambda b: (b, 0, 0)),
            pl.BlockSpec((1, KOUT, 4), lambda b: (b, 0, 0)),
            pl.BlockSpec((1, 1, KOUT), lambda b: (b, 0, 0)),
        ],
        out_shape=[
            jax.ShapeDtypeStruct((B, 1, KOUT), jnp.int32),
            jax.ShapeDtypeStruct((B, KOUT, 4), jnp.float32),
            jax.ShapeDtypeStruct((B, 1, KOUT), jnp.float32),
        ],
    )(s_t, fi_t, cand_boxes, scale)


def _debug_kernel_a_only(pred_logits, pred_boxes, orig_target_sizes):
    x_rows = pred_logits.reshape(B, NROW, NLANE)
    m = _group_max(x_rows).reshape(B, NGRP)
    _, gids = jax.lax.top_k(m, KGRP)
    scores_full = jax.nn.sigmoid(pred_logits.reshape(B, FLAT))
    cmask = jnp.zeros((B, NGRP), jnp.float32).at[
        jnp.arange(B)[:, None], gids].set(1.0)
    cmask = jnp.repeat(cmask, 8, axis=1)
    masked = jnp.where(cmask > 0, scores_full, -1.0)
    topk_scores, index = jax.lax.top_k(masked, KOUT)
    labels = index % NCLS
    box_idx = index // NCLS
    cx = pred_boxes[..., 0]
    cy = pred_boxes[..., 1]
    w = pred_boxes[..., 2]
    h = pred_boxes[..., 3]
    bbox = jnp.stack(
        [cx - 0.5 * w, cy - 0.5 * h, cx + 0.5 * w, cy + 0.5 * h], axis=-1)
    scale = jnp.tile(orig_target_sizes.astype(jnp.float32), (1, 2))[:, None, :]
    bbox = bbox * scale
    boxes = jnp.take_along_axis(
        bbox, jnp.broadcast_to(box_idx[:, :, None], (B, KOUT, 4)), axis=1)
    return (labels, boxes, topk_scores)


def _debug_kernel_ab(pred_logits, pred_boxes, orig_target_sizes):
    x_rows = pred_logits.reshape(B, NROW, NLANE)
    box_rows = pred_boxes.reshape(B, BOXROW, NLANE)
    m = _group_max(x_rows).reshape(B, NGRP)
    _, gids = jax.lax.top_k(m, KGRP)
    gids = jnp.sort(gids, axis=1)
    cand_logits, cand_boxes = _gather_candidates(x_rows, box_rows, gids)
    scores = jax.nn.sigmoid(cand_logits).reshape(B, NCAND)
    flat_idx = (gids[:, :, None] * 8
                + jnp.arange(8, dtype=jnp.int32)).reshape(B, NCAND)
    topk_scores, pos = jax.lax.top_k(scores, KOUT)
    fi = jnp.take_along_axis(flat_idx, pos, axis=1)
    labels = fi % NCLS
    raw = jnp.take_along_axis(
        cand_boxes, jnp.broadcast_to((pos // 8)[:, :, None], (B, KOUT, 4)),
        axis=1)
    cx, cy, w, h = raw[..., 0], raw[..., 1], raw[..., 2], raw[..., 3]
    bbox = jnp.stack(
        [cx - 0.5 * w, cy - 0.5 * h, cx + 0.5 * w, cy + 0.5 * h], axis=-1)
    scale = jnp.tile(orig_target_sizes.astype(jnp.float32), (1, 2))[:, None, :]
    return (labels, bbox * scale, topk_scores)


def kernel(pred_logits, pred_boxes, orig_target_sizes):
    return _debug_kernel_ab(pred_logits, pred_boxes, orig_target_sizes)
    x_rows = pred_logits.reshape(B, NROW, NLANE)
    box_rows = pred_boxes.reshape(B, BOXROW, NLANE)
    m = _group_max(x_rows).reshape(B, NGRP)
    _, gids = jax.lax.top_k(m, KGRP)
    cand_logits, cand_boxes = _gather_candidates(x_rows, box_rows, gids)
    scores = jax.nn.sigmoid(cand_logits)
    flat_idx = gids[:, :, None] * 8 + jnp.arange(8, dtype=jnp.int32)
    s_t = scores.reshape(B, 1, NCAND)
    fi_t = flat_idx.reshape(B, 1, NCAND).astype(jnp.float32)
    scale = jnp.tile(
        orig_target_sizes.astype(jnp.float32), (1, 2)).reshape(B, 1, 4)
    labels, boxes, topk_scores = _select(s_t, fi_t, cand_boxes, scale)
    return (labels.reshape(B, KOUT), boxes, topk_scores.reshape(B, KOUT))


# speedup vs baseline: 7.9492x; 1.1887x over previous
import functools

import jax
import jax.numpy as jnp
from jax.experimental import pallas as pl
from jax.experimental.pallas import tpu as pltpu

B = 16
NQ = 20000
NCLS = 80
FLAT = NQ * NCLS
NROW = 12500
NLANE = 128
GS = 16
NGRP = FLAT // GS
KGRP = 512
K1 = 320
NCAND = KGRP * GS
NSLOT = 512
KOUT = 300
BOXROW = NQ * 4 // NLANE


def _f32(x):
    return x.astype(jnp.float32)


def _amax_body(x_ref, o_ref):
    x = x_ref[0]
    w = jnp.maximum(x, jnp.concatenate([x[:, 8:], x[:, :8]], axis=1))
    w = jnp.maximum(w, jnp.concatenate([w[:, 4:], w[:, :4]], axis=1))
    w = jnp.maximum(w, jnp.concatenate([w[:, 2:], w[:, :2]], axis=1))
    w = jnp.maximum(w, jnp.concatenate([w[:, 1:], w[:, :1]], axis=1))
    li = jax.lax.broadcasted_iota(jnp.int32, (NLANE, NLANE // GS), 0)
    ci = jax.lax.broadcasted_iota(jnp.int32, (NLANE, NLANE // GS), 1)
    sel = _f32(li == GS * ci)
    o_ref[0] = jax.lax.dot_general(
        w, sel, (((1,), (0,)), ((), ())), preferred_element_type=jnp.float32)


def _group_max(x_rows):
    return pl.pallas_call(
        _amax_body,
        grid=(B,),
        in_specs=[pl.BlockSpec((1, NROW, NLANE), lambda b: (b, 0, 0))],
        out_specs=pl.BlockSpec((1, NROW, NLANE // GS), lambda b: (b, 0, 0)),
        out_shape=jax.ShapeDtypeStruct((B, NROW, NLANE // GS), jnp.float32),
    )(x_rows)


def _gather_body(gsm, x_ref, gv_ref, bx_ref, oc_ref, ob_ref, rbuf, bbuf):
    b = pl.program_id(0)

    def body(i, _):
        g = gsm[b * KGRP + i]
        rbuf[pl.ds(i, 1), :] = x_ref[0, pl.ds(g // (NLANE // GS), 1), :]
        q = g // (NCLS // GS)
        bbuf[pl.ds(i, 1), :] = bx_ref[0, pl.ds(q // 32, 1), :]
        return 0

    jax.lax.fori_loop(0, KGRP, body, 0)
    gcol = gv_ref[0]
    rows = rbuf[...]
    pcol = gcol % (NLANE // GS)
    acc = rows[:, 0:GS]
    for p in range(1, NLANE // GS):
        acc = jnp.where(pcol == p, rows[:, GS * p:GS * p + GS], acc)
    oc_ref[0] = acc
    brows = bbuf[...]
    qcol = (gcol // (NCLS // GS)) % 32
    bacc = brows[:, 0:4]
    for p in range(1, 32):
        bacc = jnp.where(qcol == p, brows[:, 4 * p:4 * p + 4], bacc)
    ob_ref[0] = bacc


def _gather_candidates(x_rows, box_rows, gids):
    grid_spec = pltpu.PrefetchScalarGridSpec(
        num_scalar_prefetch=1,
        grid=(B,),
        in_specs=[
            pl.BlockSpec((1, NROW, NLANE), lambda b, g: (b, 0, 0)),
            pl.BlockSpec((1, KGRP, 1), lambda b, g: (b, 0, 0)),
            pl.BlockSpec((1, BOXROW, NLANE), lambda b, g: (b, 0, 0)),
        ],
        out_specs=[
            pl.BlockSpec((1, KGRP, GS), lambda b, g: (b, 0, 0)),
            pl.BlockSpec((1, KGRP, 4), lambda b, g: (b, 0, 0)),
        ],
        scratch_shapes=[
            pltpu.VMEM((KGRP, NLANE), jnp.float32),
            pltpu.VMEM((KGRP, NLANE), jnp.float32),
        ],
    )
    return pl.pallas_call(
        _gather_body,
        grid_spec=grid_spec,
        out_shape=[
            jax.ShapeDtypeStruct((B, KGRP, GS), jnp.float32),
            jax.ShapeDtypeStruct((B, KGRP, 4), jnp.float32),
        ],
    )(gids.reshape(-1), x_rows, gids.reshape(B, KGRP, 1), box_rows)


def _gather_rows_body(gsm, x_ref, o_ref):
    b = pl.program_id(0)

    def body(i, _):
        g = gsm[b * KGRP + i]
        o_ref[0, pl.ds(i, 1), :] = x_ref[0, pl.ds(g // 16, 1), :]
        return 0

    jax.lax.fori_loop(0, KGRP, body, 0)


def _debug_gather_rows(x_rows, gids):
    grid_spec = pltpu.PrefetchScalarGridSpec(
        num_scalar_prefetch=1,
        grid=(B,),
        in_specs=[pl.BlockSpec((1, NROW, NLANE), lambda b, g: (b, 0, 0))],
        out_specs=pl.BlockSpec((1, KGRP, NLANE), lambda b, g: (b, 0, 0)),
    )
    return pl.pallas_call(
        _gather_rows_body,
        grid_spec=grid_spec,
        out_shape=jax.ShapeDtypeStruct((B, KGRP, NLANE), jnp.float32),
    )(gids.reshape(-1), x_rows)


def _dot(a, b):
    return jax.lax.dot_general(
        a, b, (((1,), (0,)), ((), ())), preferred_element_type=jnp.float32)


def _dot_rt(a, b):
    return jax.lax.dot_general(
        a, b, (((1,), (1,)), ((), ())), preferred_element_type=jnp.float32)


def _bc(x, shape):
    return jnp.broadcast_to(x, shape)


def _select_body(s_ref, fi_ref, cb_ref, sc_ref, lab_ref, box_ref, sco_ref):
    s = s_ref[0]
    fi = fi_ref[0]
    w = jnp.maximum(s, jnp.concatenate([s[:, 4:], s[:, :4]], axis=1))
    w = jnp.maximum(w, jnp.concatenate([w[:, 2:], w[:, :2]], axis=1))
    w = jnp.maximum(w, jnp.concatenate([w[:, 1:], w[:, :1]], axis=1))
    li = jax.lax.broadcasted_iota(jnp.int32, (NCAND, KGRP), 0)
    ci = jax.lax.broadcasted_iota(jnp.int32, (NCAND, KGRP), 1)
    sel = _f32(li == 8 * ci)
    selt = _f32(
        jax.lax.broadcasted_iota(jnp.int32, (KGRP, NCAND), 1)
        == 8 * jax.lax.broadcasted_iota(jnp.int32, (KGRP, NCAND), 0))
    m_row = _dot(w, sel)
    m_col = _dot_rt(selt, w)
    ones_r512 = jnp.ones((1, KGRP), jnp.float32)
    ones_c512 = jnp.ones((KGRP, 1), jnp.float32)
    p1 = _f32(_dot(m_col, ones_r512) > _dot(ones_c512, m_row))
    cnt = jnp.sum(p1, axis=0, keepdims=True)
    t = jnp.min(jnp.where(cnt < K1, m_row, jnp.inf))
    keep = _f32(s >= t)
    pl.debug_print("DBG w_max_mil={}", (jnp.max(w) * 1000.0).astype(jnp.int32))
    pl.debug_print("DBG m_max_mil={}", (jnp.max(m_row) * 1000.0).astype(jnp.int32))
    pl.debug_print("DBG mcol_max_mil={}", (jnp.max(m_col) * 1000.0).astype(jnp.int32))
    pl.debug_print("DBG cnt_max={}", jnp.max(cnt).astype(jnp.int32))
    pl.debug_print("DBG t_mil={}", (t * 1000.0).astype(jnp.int32))
    pl.debug_print("DBG nkeep={}", jnp.sum(keep).astype(jnp.int32))
    cs = keep
    lane = jax.lax.broadcasted_iota(jnp.int32, (1, NCAND), 1)
    sh = 1
    while sh < NCAND:
        rolled = jnp.concatenate(
            [cs[:, NCAND - sh:], cs[:, :NCAND - sh]], axis=1)
        cs = cs + jnp.where(lane >= sh, rolled, 0.0)
        sh *= 2
    pos = cs - 1.0
    ones_cs = jnp.ones((NSLOT, 1), jnp.float32)
    pos2 = _dot(ones_cs, pos).astype(jnp.int32)
    keep2 = _dot(ones_cs, keep)
    slot2 = jax.lax.broadcasted_iota(jnp.int32, (NSLOT, NCAND), 0)
    oht = jnp.where(keep2 > 0.0, _f32(slot2 == pos2), 0.0)
    pl.debug_print("DBG cs_max={}", jnp.max(cs).astype(jnp.int32))
    pl.debug_print("DBG oht_sum={}", jnp.sum(oht).astype(jnp.int32))
    sc_row = _dot_rt(s, oht)
    sc_col = _dot_rt(oht, s)
    fi_row = _dot_rt(fi, oht)
    fi_col = _dot_rt(oht, fi)
    giota = _f32(jax.lax.broadcasted_iota(jnp.int32, (1, NCAND), 1) // 8)
    g_col = _dot_rt(oht, giota)
    gj = jax.lax.broadcasted_iota(jnp.int32, (NSLOT, KGRP), 1)
    oh2 = _f32(_dot(g_col, ones_r512).astype(jnp.int32) == gj)
    bslot = _dot(oh2, cb_ref[0])
    sc_c2 = _dot(sc_col, ones_r512)
    sc_r2 = _dot(ones_c512, sc_row)
    fi_c2 = _dot(fi_col, ones_r512)
    fi_r2 = _dot(ones_c512, fi_row)
    gt = _f32(sc_c2 > sc_r2)
    tie = jnp.where(sc_c2 == sc_r2, _f32(fi_c2 < fi_r2), 0.0)
    rank_row = jnp.sum(gt + tie, axis=0, keepdims=True)
    ones_ck = jnp.ones((KOUT, 1), jnp.float32)
    ro2 = jax.lax.broadcasted_iota(jnp.int32, (KOUT, NSLOT), 0)
    oh3t = _f32(_dot(ones_ck, rank_row).astype(jnp.int32) == ro2)
    pl.debug_print("DBG scrow_max_mil={}", (jnp.max(sc_row) * 1000.0).astype(jnp.int32))
    pl.debug_print("DBG rank_max={}", jnp.max(rank_row).astype(jnp.int32))
    pl.debug_print("DBG oh2_sum={}", jnp.sum(oh2).astype(jnp.int32))
    pl.debug_print("DBG oh3t_sum={}", jnp.sum(oh3t).astype(jnp.int32))
    sco_ref[0] = _dot_rt(sc_row, oh3t)
    lab_row = _f32(fi_row.astype(jnp.int32) % NCLS)
    lab_ref[0] = _dot_rt(lab_row, oh3t).astype(jnp.int32)
    raw = _dot(oh3t, bslot)
    cx, cy, ww, hh = raw[:, 0:1], raw[:, 1:2], raw[:, 2:3], raw[:, 3:4]
    xyxy = jnp.concatenate(
        [cx - 0.5 * ww, cy - 0.5 * hh, cx + 0.5 * ww, cy + 0.5 * hh], axis=1)
    box_ref[0] = xyxy * sc_ref[0]


def _select(s_t, fi_t, cand_boxes, scale):
    return pl.pallas_call(
        _select_body,
        grid=(B,),
        in_specs=[
            pl.BlockSpec((1, 1, NCAND), lambda b: (b, 0, 0)),
            pl.BlockSpec((1, 1, NCAND), lambda b: (b, 0, 0)),
            pl.BlockSpec((1, KGRP, 4), lambda b: (b, 0, 0)),
            pl.BlockSpec((1, 1, 4), lambda b: (b, 0, 0)),
        ],
        out_specs=[
            pl.BlockSpec((1, 1, KOUT), lambda b: (b, 0, 0)),
            pl.BlockSpec((1, KOUT, 4), lambda b: (b, 0, 0)),
            pl.BlockSpec((1, 1, KOUT), lambda b: (b, 0, 0)),
        ],
        out_shape=[
            jax.ShapeDtypeStruct((B, 1, KOUT), jnp.int32),
            jax.ShapeDtypeStruct((B, KOUT, 4), jnp.float32),
            jax.ShapeDtypeStruct((B, 1, KOUT), jnp.float32),
        ],
    )(s_t, fi_t, cand_boxes, scale)


def _debug_kernel_a_only(pred_logits, pred_boxes, orig_target_sizes):
    x_rows = pred_logits.reshape(B, NROW, NLANE)
    m = _group_max(x_rows).reshape(B, NGRP)
    _, gids = jax.lax.top_k(m, KGRP)
    scores_full = jax.nn.sigmoid(pred_logits.reshape(B, FLAT))
    cmask = jnp.zeros((B, NGRP), jnp.float32).at[
        jnp.arange(B)[:, None], gids].set(1.0)
    cmask = jnp.repeat(cmask, GS, axis=1)
    masked = jnp.where(cmask > 0, scores_full, -1.0)
    topk_scores, index = jax.lax.top_k(masked, KOUT)
    labels = index % NCLS
    box_idx = index // NCLS
    cx = pred_boxes[..., 0]
    cy = pred_boxes[..., 1]
    w = pred_boxes[..., 2]
    h = pred_boxes[..., 3]
    bbox = jnp.stack(
        [cx - 0.5 * w, cy - 0.5 * h, cx + 0.5 * w, cy + 0.5 * h], axis=-1)
    scale = jnp.tile(orig_target_sizes.astype(jnp.float32), (1, 2))[:, None, :]
    bbox = bbox * scale
    boxes = jnp.take_along_axis(
        bbox, jnp.broadcast_to(box_idx[:, :, None], (B, KOUT, 4)), axis=1)
    return (labels, boxes, topk_scores)


def _debug_kernel_ab(pred_logits, pred_boxes, orig_target_sizes):
    x_rows = pred_logits.reshape(B, NROW, NLANE)
    box_rows = pred_boxes.reshape(B, BOXROW, NLANE)
    m = _group_max(x_rows).reshape(B, NGRP)
    _, gids = jax.lax.top_k(m, KGRP)
    gids = jnp.sort(gids, axis=1)
    cand_logits, cand_boxes = _gather_candidates(x_rows, box_rows, gids)
    scores = jax.nn.sigmoid(cand_logits).reshape(B, NCAND)
    flat_idx = (gids[:, :, None] * GS
                + jnp.arange(GS, dtype=jnp.int32)).reshape(B, NCAND)
    topk_scores, pos = jax.lax.top_k(scores, KOUT)
    fi = jnp.take_along_axis(flat_idx, pos, axis=1)
    labels = fi % NCLS
    raw = jnp.take_along_axis(
        cand_boxes, jnp.broadcast_to((pos // GS)[:, :, None], (B, KOUT, 4)),
        axis=1)
    cx, cy, w, h = raw[..., 0], raw[..., 1], raw[..., 2], raw[..., 3]
    bbox = jnp.stack(
        [cx - 0.5 * w, cy - 0.5 * h, cx + 0.5 * w, cy + 0.5 * h], axis=-1)
    scale = jnp.tile(orig_target_sizes.astype(jnp.float32), (1, 2))[:, None, :]
    return (labels, bbox * scale, topk_scores)


def kernel(pred_logits, pred_boxes, orig_target_sizes):
    return _debug_kernel_ab(pred_logits, pred_boxes, orig_target_sizes)
    x_rows = pred_logits.reshape(B, NROW, NLANE)
    box_rows = pred_boxes.reshape(B, BOXROW, NLANE)
    m = _group_max(x_rows).reshape(B, NGRP)
    _, gids = jax.lax.top_k(m, KGRP)
    cand_logits, cand_boxes = _gather_candidates(x_rows, box_rows, gids)
    scores = jax.nn.sigmoid(cand_logits)
    flat_idx = gids[:, :, None] * GS + jnp.arange(GS, dtype=jnp.int32)
    s_t = scores.reshape(B, 1, NCAND)
    fi_t = flat_idx.reshape(B, 1, NCAND).astype(jnp.float32)
    scale = jnp.tile(
        orig_target_sizes.astype(jnp.float32), (1, 2)).reshape(B, 1, 4)
    labels, boxes, topk_scores = _select(s_t, fi_t, cand_boxes, scale)
    return (labels.reshape(B, KOUT), boxes, topk_scores.reshape(B, KOUT))
